# baseline (speedup 1.0000x reference)
import jax
import jax.numpy as jnp
from jax import lax
from jax.experimental import pallas as pl

N = 20000
C = 20
TOPK = 2000
PRE_NMS_THRESH = 0.001
IOU_THRESHOLD = 0.5

ROW_BLK = 256

SEL_ROWS = 400
SEL_COLS = (N * C) // SEL_ROWS
SEL_K = 32


def _nms_body(s_col_ref, e_col_ref, cls_col_ref, val_col_ref,
              s_row_ref, e_row_ref, cls_row_ref, score_ref):
    i = pl.program_id(0)
    s_r = s_col_ref[...]
    e_r = e_col_ref[...]
    cls_r = cls_col_ref[...]
    val_r = val_col_ref[...]
    s_c = s_row_ref[...]
    e_c = e_row_ref[...]
    cls_c = cls_row_ref[...]

    inter = jnp.minimum(e_r, e_c) - jnp.maximum(s_r, s_c)
    inter = jnp.maximum(inter, 0.0)
    union = (e_r - s_r) + (e_c - s_c) - inter
    overlap = inter / (union + 1e-8) > IOU_THRESHOLD

    same = cls_r == cls_c
    col = jax.lax.broadcasted_iota(jnp.int32, (ROW_BLK, TOPK), 1)
    row = jax.lax.broadcasted_iota(jnp.int32, (ROW_BLK, TOPK), 0) + i * ROW_BLK
    sup = jnp.any(overlap & same & (col < row), axis=1, keepdims=True)
    keep = (~sup) & (val_r > 0.0)
    score_ref[...] = jnp.where(keep, val_r, 0.0)


def _nms_pallas(s, e, cls, vals):
    pad = -(-TOPK // ROW_BLK) * ROW_BLK
    grid = pad // ROW_BLK

    def col(x):
        return jnp.pad(x, (0, pad - TOPK)).reshape(pad, 1)

    col_spec = pl.BlockSpec((ROW_BLK, 1), lambda i: (i, 0))
    row_spec = pl.BlockSpec((1, TOPK), lambda i: (0, 0))
    out = pl.pallas_call(
        _nms_body,
        grid=(grid,),
        in_specs=[col_spec, col_spec, col_spec, col_spec,
                  row_spec, row_spec, row_spec],
        out_specs=pl.BlockSpec((ROW_BLK, 1), lambda i: (i, 0)),
        out_shape=jax.ShapeDtypeStruct((pad, 1), jnp.float32),
    )(col(s), col(e), col(cls), col(vals),
      s.reshape(1, TOPK), e.reshape(1, TOPK), cls.reshape(1, TOPK))
    return out[:TOPK].reshape(TOPK)


def kernel(segments, scores):
    flat = scores.reshape(-1)
    masked = jnp.where(flat > PRE_NMS_THRESH, flat, -1.0)

    rows = masked.reshape(SEL_ROWS, SEL_COLS)
    rvals, rcols = lax.top_k(rows, SEL_K)
    rbase = (jnp.arange(SEL_ROWS, dtype=jnp.int32) * SEL_COLS)[:, None]
    rflat = rbase + rcols
    mvals, mpos = lax.top_k(rvals.reshape(-1), TOPK)
    midx = jnp.take(rflat.reshape(-1), mpos)
    ok = jnp.all(rvals[:, -1] < mvals[-1])

    def fast(_):
        return mvals, midx

    def slow(_):
        tv, ti = lax.top_k(masked, TOPK)
        return tv, ti

    top_vals, topk_idxs = lax.cond(ok, fast, slow, None)
    pt_idxs = topk_idxs // C
    cls_idxs = topk_idxs % C
    seg = jnp.take(segments, pt_idxs, axis=0)
    score_out = _nms_pallas(seg[:, 0], seg[:, 1], cls_idxs, top_vals)
    return seg, score_out, cls_idxs


# speedup vs baseline: 1.0356x; 1.0356x over previous
import jax
import jax.numpy as jnp
from jax import lax
from jax.experimental import pallas as pl

N = 20000
C = 20
TOPK = 2000
PRE_NMS_THRESH = 0.001
IOU_THRESHOLD = 0.5

ROW_BLK = 256

SEL_ROWS = 128
SEL_COLS = (N * C) // SEL_ROWS
SEL_K = 48


def _nms_body(s_col_ref, e_col_ref, cls_col_ref, val_col_ref,
              s_row_ref, e_row_ref, cls_row_ref, score_ref):
    i = pl.program_id(0)
    s_r = s_col_ref[...]
    e_r = e_col_ref[...]
    cls_r = cls_col_ref[...]
    val_r = val_col_ref[...]
    s_c = s_row_ref[...]
    e_c = e_row_ref[...]
    cls_c = cls_row_ref[...]

    inter = jnp.minimum(e_r, e_c) - jnp.maximum(s_r, s_c)
    inter = jnp.maximum(inter, 0.0)
    union = (e_r - s_r) + (e_c - s_c) - inter
    overlap = inter / (union + 1e-8) > IOU_THRESHOLD

    same = cls_r == cls_c
    col = jax.lax.broadcasted_iota(jnp.int32, (ROW_BLK, TOPK), 1)
    row = jax.lax.broadcasted_iota(jnp.int32, (ROW_BLK, TOPK), 0) + i * ROW_BLK
    sup = jnp.any(overlap & same & (col < row), axis=1, keepdims=True)
    keep = (~sup) & (val_r > 0.0)
    score_ref[...] = jnp.where(keep, val_r, 0.0)


def _nms_pallas(s, e, cls, vals):
    pad = -(-TOPK // ROW_BLK) * ROW_BLK
    grid = pad // ROW_BLK

    def col(x):
        return jnp.pad(x, (0, pad - TOPK)).reshape(pad, 1)

    col_spec = pl.BlockSpec((ROW_BLK, 1), lambda i: (i, 0))
    row_spec = pl.BlockSpec((1, TOPK), lambda i: (0, 0))
    out = pl.pallas_call(
        _nms_body,
        grid=(grid,),
        in_specs=[col_spec, col_spec, col_spec, col_spec,
                  row_spec, row_spec, row_spec],
        out_specs=pl.BlockSpec((ROW_BLK, 1), lambda i: (i, 0)),
        out_shape=jax.ShapeDtypeStruct((pad, 1), jnp.float32),
    )(col(s), col(e), col(cls), col(vals),
      s.reshape(1, TOPK), e.reshape(1, TOPK), cls.reshape(1, TOPK))
    return out[:TOPK].reshape(TOPK)


def kernel(segments, scores):
    flat = scores.reshape(-1)
    masked = jnp.where(flat > PRE_NMS_THRESH, flat, -1.0)

    rows = masked.reshape(SEL_ROWS, SEL_COLS)
    rvals, rcols = lax.top_k(rows, SEL_K)
    rbase = (jnp.arange(SEL_ROWS, dtype=jnp.int32) * SEL_COLS)[:, None]
    rflat = rbase + rcols
    mvals, mpos = lax.top_k(rvals.reshape(-1), TOPK)
    midx = jnp.take(rflat.reshape(-1), mpos)
    ok = jnp.all(rvals[:, -1] < mvals[-1])

    def fast(_):
        return mvals, midx

    def slow(_):
        tv, ti = lax.top_k(masked, TOPK)
        return tv, ti

    top_vals, topk_idxs = lax.cond(ok, fast, slow, None)
    pt_idxs = topk_idxs // C
    cls_idxs = topk_idxs % C
    seg = jnp.take(segments, pt_idxs, axis=0)
    score_out = _nms_pallas(seg[:, 0], seg[:, 1], cls_idxs, top_vals)
    return seg, score_out, cls_idxs
